# Initial kernel scaffold; baseline (speedup 1.0000x reference)
import jax
import jax.numpy as jnp
from jax.experimental import pallas as pl


def kernel(x, edge_index, Wl1, Wr1, b1, Wl2, Wr2, b2, Wl3, Wr3, b3, Wc, bc):
    raise NotImplementedError("write your pallas kernel here")


# speedup vs baseline: 12.9889x; 12.9889x over previous
import jax
import jax.numpy as jnp
from jax import lax
from jax.experimental import pallas as pl
from jax.experimental.pallas import tpu as pltpu
from jax.experimental.pallas import tpu_sc as plsc

_N = 10000
_E = 320000
_NC = 2
_NS = 16
_NW = _NC * _NS
_CHUNK = 128
_EPT = _E // _NW
_CHUNKS = 80
_EPT_PAD = _CHUNKS * _CHUNK
_NPAD = 10240
_ROWS = _NPAD // _NS
_W = 8


def _sc_segsum(table, src3, dst3, zeros):
    out_type = [jax.ShapeDtypeStruct((_NC, _NPAD, _W), jnp.float32)]
    scratch = [
        pltpu.VMEM((_CHUNKS, _CHUNK), jnp.int32),
        pltpu.VMEM((_CHUNKS, _CHUNK), jnp.int32),
        pltpu.VMEM((2, _CHUNK, _W), jnp.float32),
        pltpu.VMEM((_ROWS, _W), jnp.float32),
        pltpu.VMEM_SHARED((_NPAD, _W), jnp.float32),
        pltpu.SemaphoreType.DMA,
        pltpu.SemaphoreType.DMA,
    ]

    def body(tab_h, src_h, dst_h, z_h, out_h, srcv, dstv, rows, zrow, acc,
             sem0, sem1):
        cid = lax.axis_index("c")
        sid = lax.axis_index("s")
        wid = cid * _NS + sid
        pltpu.sync_copy(src_h.at[wid], srcv)
        pltpu.sync_copy(dst_h.at[wid], dstv)
        pltpu.sync_copy(z_h, zrow)
        pltpu.sync_copy(zrow, acc.at[pl.ds(sid * _ROWS, _ROWS)])
        plsc.subcore_barrier()

        sems = (sem0, sem1)

        def fire(j, b):
            pltpu.async_copy(tab_h.at[srcv.at[j]], rows.at[b], sems[b])

        def drain(j, b):
            pltpu.make_async_copy(tab_h.at[srcv.at[j]], rows.at[b], sems[b]).wait()

        fire(0, 0)
        fire(1, 1)

        def step(jj, carry):
            j0 = 2 * jj
            for b in range(2):
                j = j0 + b
                drain(j, b)
                pltpu.sync_copy(rows.at[b], acc.at[dstv.at[j]], add=True)

                @pl.when(j + 2 < _CHUNKS)
                def _():
                    fire(j + 2, b)
            return carry

        lax.fori_loop(0, _CHUNKS // 2, step, 0)
        plsc.subcore_barrier()
        sl = pl.ds(sid * _ROWS, _ROWS)
        pltpu.sync_copy(acc.at[sl], out_h.at[cid, sl])

    mesh = plsc.VectorSubcoreMesh(core_axis_name="c", subcore_axis_name="s")
    fn = pl.kernel(
        body, out_type=out_type, mesh=mesh, scratch_types=scratch,
        compiler_params=pltpu.CompilerParams(use_tc_tiling_on_sc=False))
    return fn(table, src3, dst3, zeros)[0]


def _tc_first(x, w_cat, b):
    H = w_cat.shape[1] // 2

    def body(x_ref, w_ref, b_ref, t_ref, z_ref):
        xz = jnp.dot(x_ref[:], w_ref[:], preferred_element_type=jnp.float32)
        one = jnp.ones((_N, 1), jnp.float32)
        zero = jnp.zeros((_N, 3), jnp.float32)
        t_ref[:] = jnp.concatenate([xz[:, :H], one, zero], axis=1)
        z_ref[:] = xz[:, H:] + b_ref[:]

    return pl.pallas_call(
        body,
        out_shape=[jax.ShapeDtypeStruct((_N, _W), jnp.float32),
                   jax.ShapeDtypeStruct((_N, H), jnp.float32)],
    )(x, w_cat, b)


def _tc_deg_mid(p, z_prev, w_cat, b):
    Hn = w_cat.shape[1] // 2

    def body(p_ref, z_ref, w_ref, b_ref, h_ref, t_ref, zo_ref, di_ref):
        deg = p_ref[0, :_N, 4] + p_ref[1, :_N, 4]
        di = (1.0 / jnp.maximum(deg, 1.0))[:, None]
        di_ref[:] = di
        s = p_ref[0, :_N, :4] + p_ref[1, :_N, :4]
        h = jnp.tanh(s * di + z_ref[:])
        h_ref[:] = h
        yz = jnp.dot(h, w_ref[:], preferred_element_type=jnp.float32)
        pad = jnp.zeros((_N, _W - Hn), jnp.float32)
        t_ref[:] = jnp.concatenate([yz[:, :Hn], pad], axis=1)
        zo_ref[:] = yz[:, Hn:] + b_ref[:]

    return pl.pallas_call(
        body,
        out_shape=[jax.ShapeDtypeStruct((_N, 4), jnp.float32),
                   jax.ShapeDtypeStruct((_N, _W), jnp.float32),
                   jax.ShapeDtypeStruct((_N, Hn), jnp.float32),
                   jax.ShapeDtypeStruct((_N, 1), jnp.float32)],
    )(p, z_prev, w_cat, b)


def _tc_mid(p, z_prev, w_cat, b, di):
    Hn = w_cat.shape[1] // 2
    H = z_prev.shape[1]

    def body(p_ref, z_ref, w_ref, b_ref, di_ref, h_ref, t_ref, zo_ref):
        s = p_ref[0, :_N, :H] + p_ref[1, :_N, :H]
        h = jnp.tanh(s * di_ref[:] + z_ref[:])
        h_ref[:] = h
        yz = jnp.dot(h, w_ref[:], preferred_element_type=jnp.float32)
        pad = jnp.zeros((_N, _W - Hn), jnp.float32)
        t_ref[:] = jnp.concatenate([yz[:, :Hn], pad], axis=1)
        zo_ref[:] = yz[:, Hn:] + b_ref[:]

    return pl.pallas_call(
        body,
        out_shape=[jax.ShapeDtypeStruct((_N, H), jnp.float32),
                   jax.ShapeDtypeStruct((_N, _W), jnp.float32),
                   jax.ShapeDtypeStruct((_N, Hn), jnp.float32)],
    )(p, z_prev, w_cat, b, di)


def _tc_last(p, z_prev, wc, bc, di):
    C = wc.shape[1]

    def body(p_ref, z_ref, w_ref, b_ref, di_ref, h_ref, o_ref):
        s = p_ref[0, :_N, :2] + p_ref[1, :_N, :2]
        h = jnp.tanh(s * di_ref[:] + z_ref[:])
        h_ref[:] = h
        o_ref[:] = jnp.dot(h, w_ref[:], preferred_element_type=jnp.float32) + b_ref[:]

    return pl.pallas_call(
        body,
        out_shape=[jax.ShapeDtypeStruct((_N, 2), jnp.float32),
                   jax.ShapeDtypeStruct((_N, C), jnp.float32)],
    )(p, z_prev, wc, bc, di)


def kernel(x, edge_index, Wl1, Wr1, b1, Wl2, Wr2, b2, Wl3, Wr3, b3, Wc, bc):
    src, dst = edge_index[0], edge_index[1]
    src3 = jnp.pad(src.reshape(_NW, _EPT), ((0, 0), (0, _EPT_PAD - _EPT))
                   ).reshape(_NW, _CHUNKS, _CHUNK)
    dst3 = jnp.pad(dst.reshape(_NW, _EPT), ((0, 0), (0, _EPT_PAD - _EPT)),
                   constant_values=_N).reshape(_NW, _CHUNKS, _CHUNK)
    zeros = jnp.zeros((_ROWS, _W), jnp.float32)

    w1 = jnp.concatenate([Wl1, Wr1], axis=1)
    w2 = jnp.concatenate([Wl2, Wr2], axis=1)
    w3 = jnp.concatenate([Wl3, Wr3], axis=1)

    t1, z1 = _tc_first(x, w1, b1.reshape(1, -1))
    p1 = _sc_segsum(t1, src3, dst3, zeros)
    h1, t2, z2, di = _tc_deg_mid(p1, z1, w2, b2.reshape(1, -1))
    p2 = _sc_segsum(t2, src3, dst3, zeros)
    h2, t3, z3 = _tc_mid(p2, z2, w3, b3.reshape(1, -1), di)
    p3 = _sc_segsum(t3, src3, dst3, zeros)
    h3, out = _tc_last(p3, z3, Wc, bc.reshape(1, -1), di)
    return (h1, h2, h3, out)


# speedup vs baseline: 15.5480x; 1.1970x over previous
import jax
import jax.numpy as jnp
from jax import lax
from jax.experimental import pallas as pl
from jax.experimental.pallas import tpu as pltpu
from jax.experimental.pallas import tpu_sc as plsc

_N = 10000
_E = 320000
_NC = 2
_NS = 16
_NW = _NC * _NS
_CHUNK = 128
_EPT = _E // _NW
_CHUNKS = 80
_EPT_PAD = _CHUNKS * _CHUNK
_NPAD = 10240
_ROWS = _NPAD // _NS
_W = 8


def _sc_segsum(table, src3, dst3, zeros):
    NB = 8
    LAG = 4
    out_type = [jax.ShapeDtypeStruct((_NC, _NPAD, _W), jnp.float32)]
    scratch = [
        pltpu.VMEM((_CHUNKS, _CHUNK), jnp.int32),
        pltpu.VMEM((_CHUNKS, _CHUNK), jnp.int32),
        pltpu.VMEM((NB, _CHUNK, _W), jnp.float32),
        pltpu.VMEM((_ROWS, _W), jnp.float32),
        pltpu.VMEM_SHARED((_NPAD, _W), jnp.float32),
        [pltpu.SemaphoreType.DMA] * NB,
        [pltpu.SemaphoreType.DMA] * NB,
    ]

    def body(tab_h, src_h, dst_h, z_h, out_h, srcv, dstv, rows, zrow, acc,
             gsem, ssem):
        cid = lax.axis_index("c")
        sid = lax.axis_index("s")
        wid = cid * _NS + sid
        pltpu.sync_copy(src_h.at[wid], srcv)
        pltpu.sync_copy(dst_h.at[wid], dstv)
        pltpu.sync_copy(z_h, zrow)
        pltpu.sync_copy(zrow, acc.at[pl.ds(sid * _ROWS, _ROWS)])
        plsc.subcore_barrier()

        def fire_g(j, b):
            pltpu.async_copy(tab_h.at[srcv.at[j]], rows.at[b], gsem[b])

        def drain_g(j, b):
            pltpu.make_async_copy(tab_h.at[srcv.at[j]], rows.at[b], gsem[b]).wait()

        def fire_s(j, b):
            pltpu.async_copy(rows.at[b], acc.at[dstv.at[j]], ssem[b], add=True)

        def wait_s(j, b):
            pltpu.make_async_copy(rows.at[b], acc.at[dstv.at[j]], ssem[b],
                                  ).wait()

        def step(jj, carry):
            for b in range(NB):
                j = NB * jj + b

                @pl.when(jnp.logical_and(j >= NB, j < _CHUNKS))
                def _():
                    wait_s(j - NB, b)

                @pl.when(j < _CHUNKS)
                def _():
                    fire_g(j, b)

                @pl.when(jnp.logical_and(j >= LAG, j < _CHUNKS + LAG))
                def _():
                    k = j - LAG
                    bk = (b - LAG) % NB
                    drain_g(k, bk)
                    fire_s(k, bk)
            return carry

        nsteps = (_CHUNKS + LAG + NB - 1) // NB
        lax.fori_loop(0, nsteps, step, 0)
        for b in range(NB):
            j = _CHUNKS - NB + b
            wait_s(j, b)
        plsc.subcore_barrier()
        sl = pl.ds(sid * _ROWS, _ROWS)
        pltpu.sync_copy(acc.at[sl], out_h.at[cid, sl])

    mesh = plsc.VectorSubcoreMesh(core_axis_name="c", subcore_axis_name="s")
    fn = pl.kernel(
        body, out_type=out_type, mesh=mesh, scratch_types=scratch,
        compiler_params=pltpu.CompilerParams(use_tc_tiling_on_sc=False))
    return fn(table, src3, dst3, zeros)[0]


def _tc_first(x, w_cat, b):
    H = w_cat.shape[1] // 2

    def body(x_ref, w_ref, b_ref, t_ref, z_ref):
        xz = jnp.dot(x_ref[:], w_ref[:], preferred_element_type=jnp.float32)
        one = jnp.ones((_N, 1), jnp.float32)
        zero = jnp.zeros((_N, 3), jnp.float32)
        t_ref[:] = jnp.concatenate([xz[:, :H], one, zero], axis=1)
        z_ref[:] = xz[:, H:] + b_ref[:]

    return pl.pallas_call(
        body,
        out_shape=[jax.ShapeDtypeStruct((_N, _W), jnp.float32),
                   jax.ShapeDtypeStruct((_N, H), jnp.float32)],
    )(x, w_cat, b)


def _tc_deg_mid(p, z_prev, w_cat, b):
    Hn = w_cat.shape[1] // 2

    def body(p_ref, z_ref, w_ref, b_ref, h_ref, t_ref, zo_ref, di_ref):
        deg = p_ref[0, :_N, 4] + p_ref[1, :_N, 4]
        di = (1.0 / jnp.maximum(deg, 1.0))[:, None]
        di_ref[:] = di
        s = p_ref[0, :_N, :4] + p_ref[1, :_N, :4]
        h = jnp.tanh(s * di + z_ref[:])
        h_ref[:] = h
        yz = jnp.dot(h, w_ref[:], preferred_element_type=jnp.float32)
        pad = jnp.zeros((_N, _W - Hn), jnp.float32)
        t_ref[:] = jnp.concatenate([yz[:, :Hn], pad], axis=1)
        zo_ref[:] = yz[:, Hn:] + b_ref[:]

    return pl.pallas_call(
        body,
        out_shape=[jax.ShapeDtypeStruct((_N, 4), jnp.float32),
                   jax.ShapeDtypeStruct((_N, _W), jnp.float32),
                   jax.ShapeDtypeStruct((_N, Hn), jnp.float32),
                   jax.ShapeDtypeStruct((_N, 1), jnp.float32)],
    )(p, z_prev, w_cat, b)


def _tc_mid(p, z_prev, w_cat, b, di):
    Hn = w_cat.shape[1] // 2
    H = z_prev.shape[1]

    def body(p_ref, z_ref, w_ref, b_ref, di_ref, h_ref, t_ref, zo_ref):
        s = p_ref[0, :_N, :H] + p_ref[1, :_N, :H]
        h = jnp.tanh(s * di_ref[:] + z_ref[:])
        h_ref[:] = h
        yz = jnp.dot(h, w_ref[:], preferred_element_type=jnp.float32)
        pad = jnp.zeros((_N, _W - Hn), jnp.float32)
        t_ref[:] = jnp.concatenate([yz[:, :Hn], pad], axis=1)
        zo_ref[:] = yz[:, Hn:] + b_ref[:]

    return pl.pallas_call(
        body,
        out_shape=[jax.ShapeDtypeStruct((_N, H), jnp.float32),
                   jax.ShapeDtypeStruct((_N, _W), jnp.float32),
                   jax.ShapeDtypeStruct((_N, Hn), jnp.float32)],
    )(p, z_prev, w_cat, b, di)


def _tc_last(p, z_prev, wc, bc, di):
    C = wc.shape[1]

    def body(p_ref, z_ref, w_ref, b_ref, di_ref, h_ref, o_ref):
        s = p_ref[0, :_N, :2] + p_ref[1, :_N, :2]
        h = jnp.tanh(s * di_ref[:] + z_ref[:])
        h_ref[:] = h
        o_ref[:] = jnp.dot(h, w_ref[:], preferred_element_type=jnp.float32) + b_ref[:]

    return pl.pallas_call(
        body,
        out_shape=[jax.ShapeDtypeStruct((_N, 2), jnp.float32),
                   jax.ShapeDtypeStruct((_N, C), jnp.float32)],
    )(p, z_prev, wc, bc, di)


def kernel(x, edge_index, Wl1, Wr1, b1, Wl2, Wr2, b2, Wl3, Wr3, b3, Wc, bc):
    src, dst = edge_index[0], edge_index[1]
    src3 = jnp.pad(src.reshape(_NW, _EPT), ((0, 0), (0, _EPT_PAD - _EPT))
                   ).reshape(_NW, _CHUNKS, _CHUNK)
    dst3 = jnp.pad(dst.reshape(_NW, _EPT), ((0, 0), (0, _EPT_PAD - _EPT)),
                   constant_values=_N).reshape(_NW, _CHUNKS, _CHUNK)
    zeros = jnp.zeros((_ROWS, _W), jnp.float32)

    w1 = jnp.concatenate([Wl1, Wr1], axis=1)
    w2 = jnp.concatenate([Wl2, Wr2], axis=1)
    w3 = jnp.concatenate([Wl3, Wr3], axis=1)

    t1, z1 = _tc_first(x, w1, b1.reshape(1, -1))
    p1 = _sc_segsum(t1, src3, dst3, zeros)
    h1, t2, z2, di = _tc_deg_mid(p1, z1, w2, b2.reshape(1, -1))
    p2 = _sc_segsum(t2, src3, dst3, zeros)
    h2, t3, z3 = _tc_mid(p2, z2, w3, b3.reshape(1, -1), di)
    p3 = _sc_segsum(t3, src3, dst3, zeros)
    h3, out = _tc_last(p3, z3, Wc, bc.reshape(1, -1), di)
    return (h1, h2, h3, out)


# speedup vs baseline: 15.6871x; 1.0090x over previous
import jax
import jax.numpy as jnp
from jax import lax
from jax.experimental import pallas as pl
from jax.experimental.pallas import tpu as pltpu
from jax.experimental.pallas import tpu_sc as plsc

_N = 10000
_E = 320000
_NC = 2
_NS = 16
_NW = _NC * _NS
_CHUNK = 256
_EPT = _E // _NW
_CHUNKS = 40
_EPT_PAD = _CHUNKS * _CHUNK
_NPAD = 10240
_ROWS = _NPAD // _NS
_W = 8


def _sc_segsum(table, src3, dst3, zeros):
    NB = 8
    LAG = 4
    out_type = [jax.ShapeDtypeStruct((_NC, _NPAD, _W), jnp.float32)]
    scratch = [
        pltpu.VMEM((_CHUNKS, _CHUNK), jnp.int32),
        pltpu.VMEM((_CHUNKS, _CHUNK), jnp.int32),
        pltpu.VMEM((NB, _CHUNK, _W), jnp.float32),
        pltpu.VMEM((_ROWS, _W), jnp.float32),
        pltpu.VMEM_SHARED((_NPAD, _W), jnp.float32),
        [pltpu.SemaphoreType.DMA] * NB,
        [pltpu.SemaphoreType.DMA] * NB,
    ]

    def body(tab_h, src_h, dst_h, z_h, out_h, srcv, dstv, rows, zrow, acc,
             gsem, ssem):
        cid = lax.axis_index("c")
        sid = lax.axis_index("s")
        wid = cid * _NS + sid
        pltpu.sync_copy(src_h.at[wid], srcv)
        pltpu.sync_copy(dst_h.at[wid], dstv)
        pltpu.sync_copy(z_h, zrow)
        pltpu.sync_copy(zrow, acc.at[pl.ds(sid * _ROWS, _ROWS)])
        plsc.subcore_barrier()

        def fire_g(j, b):
            pltpu.async_copy(tab_h.at[srcv.at[j]], rows.at[b], gsem[b])

        def drain_g(j, b):
            pltpu.make_async_copy(tab_h.at[srcv.at[j]], rows.at[b], gsem[b]).wait()

        def fire_s(j, b):
            pltpu.async_copy(rows.at[b], acc.at[dstv.at[j]], ssem[b], add=True)

        def wait_s(j, b):
            pltpu.make_async_copy(rows.at[b], acc.at[dstv.at[j]], ssem[b],
                                  ).wait()

        def step(jj, carry):
            for b in range(NB):
                j = NB * jj + b

                @pl.when(jnp.logical_and(j >= NB, j < _CHUNKS))
                def _():
                    wait_s(j - NB, b)

                @pl.when(j < _CHUNKS)
                def _():
                    fire_g(j, b)

                @pl.when(jnp.logical_and(j >= LAG, j < _CHUNKS + LAG))
                def _():
                    k = j - LAG
                    bk = (b - LAG) % NB
                    drain_g(k, bk)
                    fire_s(k, bk)
            return carry

        nsteps = (_CHUNKS + LAG + NB - 1) // NB
        lax.fori_loop(0, nsteps, step, 0)
        for b in range(NB):
            j = _CHUNKS - NB + b
            wait_s(j, b)
        plsc.subcore_barrier()
        sl = pl.ds(sid * _ROWS, _ROWS)
        pltpu.sync_copy(acc.at[sl], out_h.at[cid, sl])

    mesh = plsc.VectorSubcoreMesh(core_axis_name="c", subcore_axis_name="s")
    fn = pl.kernel(
        body, out_type=out_type, mesh=mesh, scratch_types=scratch,
        compiler_params=pltpu.CompilerParams(use_tc_tiling_on_sc=False))
    return fn(table, src3, dst3, zeros)[0]


def _tc_first(x, w_cat, b):
    H = w_cat.shape[1] // 2

    def body(x_ref, w_ref, b_ref, t_ref, z_ref):
        xz = jnp.dot(x_ref[:], w_ref[:], preferred_element_type=jnp.float32)
        one = jnp.ones((_N, 1), jnp.float32)
        zero = jnp.zeros((_N, 3), jnp.float32)
        t_ref[:] = jnp.concatenate([xz[:, :H], one, zero], axis=1)
        z_ref[:] = xz[:, H:] + b_ref[:]

    return pl.pallas_call(
        body,
        out_shape=[jax.ShapeDtypeStruct((_N, _W), jnp.float32),
                   jax.ShapeDtypeStruct((_N, H), jnp.float32)],
    )(x, w_cat, b)


def _tc_deg_mid(p, z_prev, w_cat, b):
    Hn = w_cat.shape[1] // 2

    def body(p_ref, z_ref, w_ref, b_ref, h_ref, t_ref, zo_ref, di_ref):
        deg = p_ref[0, :_N, 4] + p_ref[1, :_N, 4]
        di = (1.0 / jnp.maximum(deg, 1.0))[:, None]
        di_ref[:] = di
        s = p_ref[0, :_N, :4] + p_ref[1, :_N, :4]
        h = jnp.tanh(s * di + z_ref[:])
        h_ref[:] = h
        yz = jnp.dot(h, w_ref[:], preferred_element_type=jnp.float32)
        pad = jnp.zeros((_N, _W - Hn), jnp.float32)
        t_ref[:] = jnp.concatenate([yz[:, :Hn], pad], axis=1)
        zo_ref[:] = yz[:, Hn:] + b_ref[:]

    return pl.pallas_call(
        body,
        out_shape=[jax.ShapeDtypeStruct((_N, 4), jnp.float32),
                   jax.ShapeDtypeStruct((_N, _W), jnp.float32),
                   jax.ShapeDtypeStruct((_N, Hn), jnp.float32),
                   jax.ShapeDtypeStruct((_N, 1), jnp.float32)],
    )(p, z_prev, w_cat, b)


def _tc_mid(p, z_prev, w_cat, b, di):
    Hn = w_cat.shape[1] // 2
    H = z_prev.shape[1]

    def body(p_ref, z_ref, w_ref, b_ref, di_ref, h_ref, t_ref, zo_ref):
        s = p_ref[0, :_N, :H] + p_ref[1, :_N, :H]
        h = jnp.tanh(s * di_ref[:] + z_ref[:])
        h_ref[:] = h
        yz = jnp.dot(h, w_ref[:], preferred_element_type=jnp.float32)
        pad = jnp.zeros((_N, _W - Hn), jnp.float32)
        t_ref[:] = jnp.concatenate([yz[:, :Hn], pad], axis=1)
        zo_ref[:] = yz[:, Hn:] + b_ref[:]

    return pl.pallas_call(
        body,
        out_shape=[jax.ShapeDtypeStruct((_N, H), jnp.float32),
                   jax.ShapeDtypeStruct((_N, _W), jnp.float32),
                   jax.ShapeDtypeStruct((_N, Hn), jnp.float32)],
    )(p, z_prev, w_cat, b, di)


def _tc_last(p, z_prev, wc, bc, di):
    C = wc.shape[1]

    def body(p_ref, z_ref, w_ref, b_ref, di_ref, h_ref, o_ref):
        s = p_ref[0, :_N, :2] + p_ref[1, :_N, :2]
        h = jnp.tanh(s * di_ref[:] + z_ref[:])
        h_ref[:] = h
        o_ref[:] = jnp.dot(h, w_ref[:], preferred_element_type=jnp.float32) + b_ref[:]

    return pl.pallas_call(
        body,
        out_shape=[jax.ShapeDtypeStruct((_N, 2), jnp.float32),
                   jax.ShapeDtypeStruct((_N, C), jnp.float32)],
    )(p, z_prev, wc, bc, di)


def kernel(x, edge_index, Wl1, Wr1, b1, Wl2, Wr2, b2, Wl3, Wr3, b3, Wc, bc):
    src, dst = edge_index[0], edge_index[1]
    src3 = jnp.pad(src.reshape(_NW, _EPT), ((0, 0), (0, _EPT_PAD - _EPT))
                   ).reshape(_NW, _CHUNKS, _CHUNK)
    dst3 = jnp.pad(dst.reshape(_NW, _EPT), ((0, 0), (0, _EPT_PAD - _EPT)),
                   constant_values=_N).reshape(_NW, _CHUNKS, _CHUNK)
    zeros = jnp.zeros((_ROWS, _W), jnp.float32)

    w1 = jnp.concatenate([Wl1, Wr1], axis=1)
    w2 = jnp.concatenate([Wl2, Wr2], axis=1)
    w3 = jnp.concatenate([Wl3, Wr3], axis=1)

    t1, z1 = _tc_first(x, w1, b1.reshape(1, -1))
    p1 = _sc_segsum(t1, src3, dst3, zeros)
    h1, t2, z2, di = _tc_deg_mid(p1, z1, w2, b2.reshape(1, -1))
    p2 = _sc_segsum(t2, src3, dst3, zeros)
    h2, t3, z3 = _tc_mid(p2, z2, w3, b3.reshape(1, -1), di)
    p3 = _sc_segsum(t3, src3, dst3, zeros)
    h3, out = _tc_last(p3, z3, Wc, bc.reshape(1, -1), di)
    return (h1, h2, h3, out)
